# baseline (speedup 1.0000x reference)
import functools

import jax
import jax.numpy as jnp
from jax import lax
from jax.experimental import pallas as pl
from jax.experimental.pallas import tpu as pltpu
from jax.experimental.pallas import tpu_sc as plsc

N = 4096
D = 1024
E = 8
H = 2816
C = 640
SB = C + 8
K = 2

NC, NS = 2, 16
NW = NC * NS
TPW = N // NW
NGD = TPW // 16
NR = NS
TPR = N // NR
NGR = TPR // 16

_NEG_INF = float("-inf")


def _logits_body(x_ref, wr_ref, out_ref):
    out_ref[...] = lax.dot_general(
        wr_ref[...], x_ref[...], (((1,), (1,)), ((), ())),
        preferred_element_type=jnp.float32)


def _logits_call(x_flat, w_router):
    return pl.pallas_call(
        _logits_body,
        grid=(8,),
        in_specs=[
            pl.BlockSpec((N // 8, D), lambda j: (j, 0)),
            pl.BlockSpec((E, D), lambda j: (0, 0)),
        ],
        out_specs=pl.BlockSpec((E, N // 8), lambda j: (0, j)),
        out_shape=jax.ShapeDtypeStruct((E, N), jnp.float32),
    )(x_flat, w_router)


def _hidden_body(x_ref, wg_ref, out_ref):
    acc = lax.dot_general(
        x_ref[...], wg_ref[...], (((1,), (1,)), ((), ())),
        preferred_element_type=jnp.float32)
    out_ref[...] = (acc * jax.nn.sigmoid(acc)).astype(jnp.bfloat16)


def _hidden_call(x_flat, w_gate):
    bn, bh = 1024, H // 2
    return pl.pallas_call(
        _hidden_body,
        grid=(H // bh, N // bn),
        in_specs=[
            pl.BlockSpec((bn, D), lambda j, i: (i, 0)),
            pl.BlockSpec((bh, D), lambda j, i: (j, 0)),
        ],
        out_specs=pl.BlockSpec((bn, bh), lambda j, i: (i, j)),
        out_shape=jax.ShapeDtypeStruct((N, H), jnp.bfloat16),
    )(x_flat, w_gate)


def _route_body(lg_hbm, idxd_hbm, idxc_hbm, sg_hbm, cnt_hbm, xch_hbm,
                lg_v, i12_v, p12_v, g12_v, ib_v, ic_v, gn_v,
                sgall_v, isg_v, cntv_v, base_v, all_v, sgsem0, sgsem1):
    rid = lax.axis_index("s")
    base = rid * TPR
    pltpu.sync_copy(lg_hbm.at[:, pl.ds(base, TPR)], lg_v)

    iota = lax.iota(jnp.int32, 16)
    carry = [jnp.int32(0)] * E

    for g in range(NGR):
        vs = [lg_v[e, pl.ds(g * 16, 16)] for e in range(E)]
        m1 = vs[0]
        for e in range(1, E):
            m1 = jnp.maximum(m1, vs[e])
        i1 = jnp.zeros((16,), jnp.int32)
        for e in range(E - 1, -1, -1):
            i1 = jnp.where(vs[e] == m1, jnp.int32(e), i1)
        den = jnp.zeros((16,), jnp.float32)
        for e in range(E):
            den = den + jnp.exp(vs[e] - m1)
        vmask = [jnp.where(i1 == e, _NEG_INF, vs[e]) for e in range(E)]
        m2 = vmask[0]
        for e in range(1, E):
            m2 = jnp.maximum(m2, vmask[e])
        i2 = jnp.zeros((16,), jnp.int32)
        for e in range(E - 1, -1, -1):
            i2 = jnp.where(vmask[e] == m2, jnp.int32(e), i2)
        g1 = 1.0 / den
        g2 = jnp.exp(m2 - m1) / den

        p1 = jnp.zeros((16,), jnp.int32)
        p2 = jnp.zeros((16,), jnp.int32)
        for e in range(E):
            sel1 = i1 == e
            sel2 = i2 == e
            me = jnp.where(sel1 | sel2, jnp.int32(1), jnp.int32(0))
            cs = plsc.cumsum(me)
            pos_e = cs + (carry[e] - 1)
            p1 = jnp.where(sel1, pos_e, p1)
            p2 = jnp.where(sel2, pos_e, p2)
            carry[e] = carry[e] + jnp.sum(me)

        i12_v[0, g] = i1
        i12_v[1, g] = i2
        p12_v[0, g] = p1
        p12_v[1, g] = p2
        g12_v[0, g] = g1
        g12_v[1, g] = g2

    cntvec = jnp.zeros((16,), jnp.int32)
    for e in range(E):
        cntvec = jnp.where(iota == e, carry[e], cntvec)
    cntv_v[...] = cntvec
    pltpu.sync_copy(cntv_v, xch_hbm.at[rid])
    plsc.subcore_barrier()
    pltpu.sync_copy(xch_hbm, all_v)
    basevec = jnp.zeros((16,), jnp.int32)
    totvec = jnp.zeros((16,), jnp.int32)
    for w in range(NR):
        row = all_v[w]
        flag = jnp.where(jnp.int32(w) < rid, jnp.int32(1), jnp.int32(0))
        basevec = basevec + row * flag
        totvec = totvec + row
    base_v[...] = basevec

    @pl.when(rid == 0)
    def _():
        cntv_v[...] = jnp.minimum(totvec, jnp.int32(C))
        pltpu.sync_copy(cntv_v, cnt_hbm)

    for g in range(NGR):
        i1 = i12_v[0, g]
        i2 = i12_v[1, g]
        b1 = plsc.load_gather(base_v, [i1])
        b2 = plsc.load_gather(base_v, [i2])
        P1 = p12_v[0, g] + b1
        P2 = p12_v[1, g] + b2
        k1 = P1 < C
        k2 = P2 < C
        g1k = jnp.where(k1, g12_v[0, g], jnp.float32(0.0))
        g2k = jnp.where(k2, g12_v[1, g], jnp.float32(0.0))
        den = g1k + g2k + jnp.float32(1e-6)
        h = g // NGD
        gg = g % NGD
        s1 = i1 * SB + jnp.minimum(P1, jnp.int32(C))
        s2 = i2 * SB + jnp.minimum(P2, jnp.int32(C))
        ib_v[h, 0, gg] = s1
        ib_v[h, 1, gg] = s2
        ic_v[h, gg, pl.ds(0, 16)] = s1
        ic_v[h, gg, pl.ds(16, 16)] = s2
        r0 = g * 2 * 16
        isg_v[r0 // 128, pl.ds(r0 % 128, 16)] = s1
        isg_v[(r0 + 16) // 128, pl.ds((r0 + 16) % 128, 16)] = s2
        gn_v[0] = g1k / den
        gn_v[1] = g2k / den
        for j in range(K):
            jj = jnp.zeros((16,), jnp.int32) + j
            for t in range(16):
                tt = jnp.zeros((16,), jnp.int32) + t
                sgall_v[r0 + j * 16 + t, pl.ds(0, 16)] = (
                    plsc.load_gather(gn_v, [jj, tt]))

    sgput = []
    for s in range(4):
        sgput.append(pltpu.async_copy(
            sgall_v.at[pl.ds(s * 128, 128)], sg_hbm.at[isg_v.at[s]],
            sgsem0))
    for hh in range(2):
        w_out = rid * 2 + hh
        pltpu.sync_copy(ib_v.at[hh], idxd_hbm.at[w_out])
        pltpu.sync_copy(ic_v.at[hh], idxc_hbm.at[w_out])
    for p in sgput:
        p.wait()


def _route_call(logitsT):
    route = functools.partial(
        pl.kernel,
        out_type=[
            jax.ShapeDtypeStruct((NW, K, NGD, 16), jnp.int32),
            jax.ShapeDtypeStruct((NW, NGD, 2 * 16), jnp.int32),
            jax.ShapeDtypeStruct((E * SB, 128), jnp.float32),
            jax.ShapeDtypeStruct((16,), jnp.int32),
            jax.ShapeDtypeStruct((NR, 16), jnp.int32),
        ],
        mesh=plsc.VectorSubcoreMesh(
            core_axis_name="c", subcore_axis_name="s", num_cores=1),
        compiler_params=pltpu.CompilerParams(needs_layout_passes=False),
        scratch_types=[
            pltpu.VMEM((E, TPR), jnp.float32),
            pltpu.VMEM((K, NGR, 16), jnp.int32),
            pltpu.VMEM((K, NGR, 16), jnp.int32),
            pltpu.VMEM((K, NGR, 16), jnp.float32),
            pltpu.VMEM((2, K, NGD, 16), jnp.int32),
            pltpu.VMEM((2, NGD, 2 * 16), jnp.int32),
            pltpu.VMEM((K, 16), jnp.float32),
            pltpu.VMEM((K * TPR, 128), jnp.float32),
            pltpu.VMEM((4, 128), jnp.int32),
            pltpu.VMEM((16,), jnp.int32),
            pltpu.VMEM((16,), jnp.int32),
            pltpu.VMEM((NR, 16), jnp.int32),
            pltpu.SemaphoreType.DMA,
            pltpu.SemaphoreType.DMA,
        ],
    )(_route_body)
    idxd, idxc, sg, cnt, _ = route(logitsT)
    return idxd, idxc, sg, cnt


def _dispatch_body(hidden_hbm, idxbuf_hbm, buf_hbm, idx_v, h0_v, h1_v,
                   lsem0, lsem1, ssem0, ssem1):
    cid = lax.axis_index("c")
    sid = lax.axis_index("s")
    wid = sid * NC + cid
    pltpu.sync_copy(idxbuf_hbm.at[wid], idx_v)
    hbufs = (h0_v, h1_v)
    lsems = (lsem0, lsem1)
    ssems = (ssem0, ssem1)
    loads = [None, None]
    scats = [None, None]

    def start_load(g):
        b = g % 2
        t0 = wid * TPW + g * 16
        loads[b] = pltpu.async_copy(
            hidden_hbm.at[pl.ds(t0, 16)], hbufs[b], lsems[b])

    start_load(0)
    for g in range(NGD):
        b = g % 2
        if g + 1 < NGD:
            if scats[1 - b] is not None:
                for s in scats[1 - b]:
                    s.wait()
                scats[1 - b] = None
            start_load(g + 1)
        loads[b].wait()
        scats[b] = (
            pltpu.async_copy(hbufs[b], buf_hbm.at[idx_v.at[0, g]], ssems[b]),
            pltpu.async_copy(hbufs[b], buf_hbm.at[idx_v.at[1, g]], ssems[b]),
        )
    for pair in scats:
        if pair is not None:
            for s in pair:
                s.wait()


def _dispatch_call(hidden, idxbuf):
    hw = hidden.shape[1]
    disp = functools.partial(
        pl.kernel,
        out_type=jax.ShapeDtypeStruct((E * SB, hw), jnp.float32),
        mesh=plsc.VectorSubcoreMesh(core_axis_name="c", subcore_axis_name="s"),
        compiler_params=pltpu.CompilerParams(needs_layout_passes=False),
        scratch_types=[
            pltpu.VMEM((K, NGD, 16), jnp.int32),
            pltpu.VMEM((16, hw), jnp.float32),
            pltpu.VMEM((16, hw), jnp.float32),
            pltpu.SemaphoreType.DMA,
            pltpu.SemaphoreType.DMA,
            pltpu.SemaphoreType.DMA,
            pltpu.SemaphoreType.DMA,
        ],
    )(_dispatch_body)
    return disp(hidden, idxbuf)


def _down_body(cnt_ref, buf_ref, wd_ref, sg_ref, out_ref):
    e = pl.program_id(0)
    acc = lax.dot_general(
        buf_ref[0], wd_ref[0], (((1,), (1,)), ((), ())),
        preferred_element_type=jnp.float32)
    gate = sg_ref[0, :, 0:1]
    rows = lax.broadcasted_iota(jnp.int32, acc.shape, 0)
    out_ref[0] = jnp.where(rows < cnt_ref[e], acc * gate, jnp.float32(0.0))


def _down_call(cnt, buffer, sg, w_down):
    bd = D // 2
    grid_spec = pltpu.PrefetchScalarGridSpec(
        num_scalar_prefetch=1,
        grid=(E, D // bd),
        in_specs=[
            pl.BlockSpec((1, SB, H), lambda e, di, cnt_ref: (e, 0, 0)),
            pl.BlockSpec((1, bd, H), lambda e, di, cnt_ref: (e, di, 0)),
            pl.BlockSpec((1, SB, 128), lambda e, di, cnt_ref: (e, 0, 0)),
        ],
        out_specs=pl.BlockSpec((1, SB, bd), lambda e, di, cnt_ref: (e, 0, di)),
    )
    buf3 = buffer.reshape(E, SB, H)
    sg3 = sg.reshape(E, SB, 128)
    return pl.pallas_call(
        _down_body,
        grid_spec=grid_spec,
        out_shape=jax.ShapeDtypeStruct((E, SB, D), jnp.float32),
    )(cnt, buf3, w_down, sg3)


def _combine_body(outbuf_hbm, idxc_hbm, y_hbm,
                  io_v, ra_v, rb_v, rc_v,
                  gsem0, gsem1, gsem2, wsem0, wsem1, wsem2):
    cid = lax.axis_index("c")
    sid = lax.axis_index("s")
    wid = sid * NC + cid
    pltpu.sync_copy(idxc_hbm.at[wid], io_v)
    rs = (ra_v, rb_v, rc_v)
    gsems = (gsem0, gsem1, gsem2)
    wsems = (wsem0, wsem1, wsem2)
    gets = [None, None, None]
    puts = [None, None, None]

    def start_gather(g):
        b = g % 3
        if puts[b] is not None:
            puts[b].wait()
            puts[b] = None
        gets[b] = pltpu.async_copy(
            outbuf_hbm.at[io_v.at[g]], rs[b], gsems[b])

    start_gather(0)
    start_gather(1)
    for g in range(NGD):
        b = g % 3
        if g + 2 < NGD:
            start_gather(g + 2)
        gets[b].wait()
        r_v = rs[b]

        def body(q, _):
            off = q * 16
            for t in range(16):
                r_v[t, pl.ds(off, 16)] = (
                    r_v[t, pl.ds(off, 16)] + r_v[16 + t, pl.ds(off, 16)])
            return 0

        lax.fori_loop(0, D // 16, body, 0)
        puts[b] = pltpu.async_copy(
            r_v.at[pl.ds(0, 16)],
            y_hbm.at[pl.ds(wid * TPW + g * 16, 16)], wsems[b])
    for p in puts:
        if p is not None:
            p.wait()


def _combine_call(out_flat, idxc):
    comb = functools.partial(
        pl.kernel,
        out_type=jax.ShapeDtypeStruct((N, D), jnp.float32),
        mesh=plsc.VectorSubcoreMesh(core_axis_name="c", subcore_axis_name="s"),
        compiler_params=pltpu.CompilerParams(needs_layout_passes=False),
        scratch_types=[
            pltpu.VMEM((NGD, 2 * 16), jnp.int32),
            pltpu.VMEM((2 * 16, D), jnp.float32),
            pltpu.VMEM((2 * 16, D), jnp.float32),
            pltpu.VMEM((2 * 16, D), jnp.float32),
            pltpu.SemaphoreType.DMA,
            pltpu.SemaphoreType.DMA,
            pltpu.SemaphoreType.DMA,
            pltpu.SemaphoreType.DMA,
            pltpu.SemaphoreType.DMA,
            pltpu.SemaphoreType.DMA,
        ],
    )(_combine_body)
    return comb(out_flat, idxc)


def kernel(x, W_router, W_gate, W_down):
    B, T, _ = x.shape
    x_flat = x.reshape(N, D)
    logitsT = _logits_call(x_flat, W_router)
    idxd, idxc, sg, cnt = _route_call(logitsT)
    hidden = _hidden_call(x_flat, W_gate)
    hv = lax.bitcast_convert_type(hidden.reshape(N, H // 2, 2), jnp.float32)
    buffer = _dispatch_call(hv, idxd)
    buf_bf = lax.bitcast_convert_type(buffer, jnp.bfloat16).reshape(E * SB, H)
    out_buf = _down_call(cnt, buf_bf, sg, W_down.astype(jnp.bfloat16))
    y = _combine_call(out_buf.reshape(E * SB, D), idxc)
    return y.reshape(B, T, D)


# speedup vs baseline: 3.0400x; 3.0400x over previous
import functools

import jax
import jax.numpy as jnp
from jax import lax
from jax.experimental import pallas as pl
from jax.experimental.pallas import tpu as pltpu
from jax.experimental.pallas import tpu_sc as plsc

N = 4096
D = 1024
E = 8
H = 2816
C = 640
SB = C + 8
K = 2

H2 = H // 2

NC, NS = 2, 16
NW = NC * NS
TPW = N // NW
NGD = TPW // 16
NR = NS
TPR = N // NR
NGR = TPR // 16

_NEG_INF = float("-inf")


def _logits_body(x_ref, wr_ref, out_ref):
    out_ref[...] = lax.dot_general(
        wr_ref[...], x_ref[...], (((1,), (1,)), ((), ())),
        preferred_element_type=jnp.float32)


def _logits_call(x_flat, w_router):
    return pl.pallas_call(
        _logits_body,
        grid=(8,),
        in_specs=[
            pl.BlockSpec((N // 8, D), lambda j: (j, 0)),
            pl.BlockSpec((E, D), lambda j: (0, 0)),
        ],
        out_specs=pl.BlockSpec((E, N // 8), lambda j: (0, j)),
        out_shape=jax.ShapeDtypeStruct((E, N), jnp.float32),
    )(x_flat, w_router)


def _hidden_body(x_ref, wg_ref, out_ref):
    xb = x_ref[...].astype(jnp.bfloat16)
    wb = wg_ref[...].astype(jnp.bfloat16)
    acc = lax.dot_general(
        xb, wb, (((1,), (1,)), ((), ())),
        preferred_element_type=jnp.float32)
    h = (acc * jax.nn.sigmoid(acc)).astype(jnp.bfloat16)
    lo = lax.bitcast_convert_type(h[:, :H2], jnp.uint16).astype(jnp.uint32)
    hi = lax.bitcast_convert_type(h[:, H2:], jnp.uint16).astype(jnp.uint32)
    out_ref[...] = lax.bitcast_convert_type(lo | (hi << 16), jnp.float32)


def _hidden_call(x_flat, w_gate):
    bn = 512
    return pl.pallas_call(
        _hidden_body,
        grid=(N // bn,),
        in_specs=[
            pl.BlockSpec((bn, D), lambda i: (i, 0)),
            pl.BlockSpec((H, D), lambda i: (0, 0)),
        ],
        out_specs=pl.BlockSpec((bn, H2), lambda i: (i, 0)),
        out_shape=jax.ShapeDtypeStruct((N, H2), jnp.float32),
    )(x_flat, w_gate)


def _route_body(lg_hbm, idxd_hbm, idxc_hbm, sg_hbm, cnt_hbm, xch_hbm,
                lg_v, i12_v, p12_v, g12_v, ib_v, ic_v, gn_v,
                sgall_v, isg_v, cntv_v, base_v, all_v, sgsem0, sgsem1):
    rid = lax.axis_index("s")
    base = rid * TPR
    pltpu.sync_copy(lg_hbm.at[:, pl.ds(base, TPR)], lg_v)

    iota = lax.iota(jnp.int32, 16)
    carry = [jnp.int32(0)] * E

    for g in range(NGR):
        vs = [lg_v[e, pl.ds(g * 16, 16)] for e in range(E)]
        m1 = vs[0]
        for e in range(1, E):
            m1 = jnp.maximum(m1, vs[e])
        i1 = jnp.zeros((16,), jnp.int32)
        for e in range(E - 1, -1, -1):
            i1 = jnp.where(vs[e] == m1, jnp.int32(e), i1)
        den = jnp.zeros((16,), jnp.float32)
        for e in range(E):
            den = den + jnp.exp(vs[e] - m1)
        vmask = [jnp.where(i1 == e, _NEG_INF, vs[e]) for e in range(E)]
        m2 = vmask[0]
        for e in range(1, E):
            m2 = jnp.maximum(m2, vmask[e])
        i2 = jnp.zeros((16,), jnp.int32)
        for e in range(E - 1, -1, -1):
            i2 = jnp.where(vmask[e] == m2, jnp.int32(e), i2)
        g1 = 1.0 / den
        g2 = jnp.exp(m2 - m1) / den

        p1 = jnp.zeros((16,), jnp.int32)
        p2 = jnp.zeros((16,), jnp.int32)
        for e in range(E):
            sel1 = i1 == e
            sel2 = i2 == e
            me = jnp.where(sel1 | sel2, jnp.int32(1), jnp.int32(0))
            cs = plsc.cumsum(me)
            pos_e = cs + (carry[e] - 1)
            p1 = jnp.where(sel1, pos_e, p1)
            p2 = jnp.where(sel2, pos_e, p2)
            carry[e] = carry[e] + jnp.sum(me)

        i12_v[0, g] = i1
        i12_v[1, g] = i2
        p12_v[0, g] = p1
        p12_v[1, g] = p2
        g12_v[0, g] = g1
        g12_v[1, g] = g2

    cntvec = jnp.zeros((16,), jnp.int32)
    for e in range(E):
        cntvec = jnp.where(iota == e, carry[e], cntvec)
    cntv_v[...] = cntvec
    pltpu.sync_copy(cntv_v, xch_hbm.at[rid])
    plsc.subcore_barrier()
    pltpu.sync_copy(xch_hbm, all_v)
    basevec = jnp.zeros((16,), jnp.int32)
    totvec = jnp.zeros((16,), jnp.int32)
    for w in range(NR):
        row = all_v[w]
        flag = jnp.where(jnp.int32(w) < rid, jnp.int32(1), jnp.int32(0))
        basevec = basevec + row * flag
        totvec = totvec + row
    base_v[...] = basevec

    @pl.when(rid == 0)
    def _():
        cntv_v[...] = jnp.minimum(totvec, jnp.int32(C))
        pltpu.sync_copy(cntv_v, cnt_hbm)

    for g in range(NGR):
        i1 = i12_v[0, g]
        i2 = i12_v[1, g]
        b1 = plsc.load_gather(base_v, [i1])
        b2 = plsc.load_gather(base_v, [i2])
        P1 = p12_v[0, g] + b1
        P2 = p12_v[1, g] + b2
        k1 = P1 < C
        k2 = P2 < C
        g1k = jnp.where(k1, g12_v[0, g], jnp.float32(0.0))
        g2k = jnp.where(k2, g12_v[1, g], jnp.float32(0.0))
        den = g1k + g2k + jnp.float32(1e-6)
        h = g // NGD
        gg = g % NGD
        s1 = i1 * SB + jnp.minimum(P1, jnp.int32(C))
        s2 = i2 * SB + jnp.minimum(P2, jnp.int32(C))
        ib_v[h, 0, gg] = s1
        ib_v[h, 1, gg] = s2
        ic_v[h, gg, pl.ds(0, 16)] = s1
        ic_v[h, gg, pl.ds(16, 16)] = s2
        r0 = g * 2 * 16
        isg_v[r0 // 128, pl.ds(r0 % 128, 16)] = s1
        isg_v[(r0 + 16) // 128, pl.ds((r0 + 16) % 128, 16)] = s2
        gn_v[0] = g1k / den
        gn_v[1] = g2k / den
        for j in range(K):
            jj = jnp.zeros((16,), jnp.int32) + j
            for t in range(16):
                tt = jnp.zeros((16,), jnp.int32) + t
                sgall_v[r0 + j * 16 + t, pl.ds(0, 16)] = (
                    plsc.load_gather(gn_v, [jj, tt]))

    sgput = []
    for s in range(4):
        sgput.append(pltpu.async_copy(
            sgall_v.at[pl.ds(s * 128, 128)], sg_hbm.at[isg_v.at[s]],
            sgsem0))
    for hh in range(2):
        w_out = rid * 2 + hh
        pltpu.sync_copy(ib_v.at[hh], idxd_hbm.at[w_out])
        pltpu.sync_copy(ic_v.at[hh], idxc_hbm.at[w_out])
    for p in sgput:
        p.wait()


def _route_call(logitsT):
    route = functools.partial(
        pl.kernel,
        out_type=[
            jax.ShapeDtypeStruct((NW, K, NGD, 16), jnp.int32),
            jax.ShapeDtypeStruct((NW, NGD, 2 * 16), jnp.int32),
            jax.ShapeDtypeStruct((E * SB, 128), jnp.float32),
            jax.ShapeDtypeStruct((16,), jnp.int32),
            jax.ShapeDtypeStruct((NR, 16), jnp.int32),
        ],
        mesh=plsc.VectorSubcoreMesh(
            core_axis_name="c", subcore_axis_name="s", num_cores=1),
        compiler_params=pltpu.CompilerParams(needs_layout_passes=False),
        scratch_types=[
            pltpu.VMEM((E, TPR), jnp.float32),
            pltpu.VMEM((K, NGR, 16), jnp.int32),
            pltpu.VMEM((K, NGR, 16), jnp.int32),
            pltpu.VMEM((K, NGR, 16), jnp.float32),
            pltpu.VMEM((2, K, NGD, 16), jnp.int32),
            pltpu.VMEM((2, NGD, 2 * 16), jnp.int32),
            pltpu.VMEM((K, 16), jnp.float32),
            pltpu.VMEM((K * TPR, 128), jnp.float32),
            pltpu.VMEM((4, 128), jnp.int32),
            pltpu.VMEM((16,), jnp.int32),
            pltpu.VMEM((16,), jnp.int32),
            pltpu.VMEM((NR, 16), jnp.int32),
            pltpu.SemaphoreType.DMA,
            pltpu.SemaphoreType.DMA,
        ],
    )(_route_body)
    idxd, idxc, sg, cnt, _ = route(logitsT)
    return idxd, idxc, sg, cnt


def _dispatch_body(hidden_hbm, idxbuf_hbm, buf_hbm, idx_v, h0_v, h1_v,
                   lsem0, lsem1, ssem0, ssem1):
    cid = lax.axis_index("c")
    sid = lax.axis_index("s")
    wid = sid * NC + cid
    pltpu.sync_copy(idxbuf_hbm.at[wid], idx_v)
    hbufs = (h0_v, h1_v)
    lsems = (lsem0, lsem1)
    ssems = (ssem0, ssem1)
    loads = [None, None]
    scats = [None, None]

    def start_load(g):
        b = g % 2
        t0 = wid * TPW + g * 16
        loads[b] = pltpu.async_copy(
            hidden_hbm.at[pl.ds(t0, 16)], hbufs[b], lsems[b])

    start_load(0)
    for g in range(NGD):
        b = g % 2
        if g + 1 < NGD:
            if scats[1 - b] is not None:
                for s in scats[1 - b]:
                    s.wait()
                scats[1 - b] = None
            start_load(g + 1)
        loads[b].wait()
        scats[b] = (
            pltpu.async_copy(hbufs[b], buf_hbm.at[idx_v.at[0, g]], ssems[b]),
            pltpu.async_copy(hbufs[b], buf_hbm.at[idx_v.at[1, g]], ssems[b]),
        )
    for pair in scats:
        if pair is not None:
            for s in pair:
                s.wait()


def _dispatch_call(hidden, idxbuf):
    hw = hidden.shape[1]
    disp = functools.partial(
        pl.kernel,
        out_type=jax.ShapeDtypeStruct((E * SB, hw), jnp.float32),
        mesh=plsc.VectorSubcoreMesh(core_axis_name="c", subcore_axis_name="s"),
        compiler_params=pltpu.CompilerParams(needs_layout_passes=False),
        scratch_types=[
            pltpu.VMEM((K, NGD, 16), jnp.int32),
            pltpu.VMEM((16, hw), jnp.float32),
            pltpu.VMEM((16, hw), jnp.float32),
            pltpu.SemaphoreType.DMA,
            pltpu.SemaphoreType.DMA,
            pltpu.SemaphoreType.DMA,
            pltpu.SemaphoreType.DMA,
        ],
    )(_dispatch_body)
    return disp(hidden, idxbuf)


def _down_body(cnt_ref, buf_ref, wd_ref, sg_ref, out_ref):
    e = pl.program_id(0)
    pu = lax.bitcast_convert_type(buf_ref[0], jnp.uint32)
    lo = lax.bitcast_convert_type(
        (pu & jnp.uint32(0xFFFF)).astype(jnp.uint16), jnp.bfloat16)
    hi = lax.bitcast_convert_type(
        (pu >> jnp.uint32(16)).astype(jnp.uint16), jnp.bfloat16)
    wd = wd_ref[0].astype(jnp.bfloat16)
    acc = lax.dot_general(
        lo, wd[:, :H2], (((1,), (1,)), ((), ())),
        preferred_element_type=jnp.float32)
    acc = acc + lax.dot_general(
        hi, wd[:, H2:], (((1,), (1,)), ((), ())),
        preferred_element_type=jnp.float32)
    gate = sg_ref[0, :, 0:1]
    rows = lax.broadcasted_iota(jnp.int32, acc.shape, 0)
    out_ref[0] = jnp.where(rows < cnt_ref[e], acc * gate, jnp.float32(0.0))


def _down_call(cnt, buffer, sg, w_down):
    bd = D // 2
    grid_spec = pltpu.PrefetchScalarGridSpec(
        num_scalar_prefetch=1,
        grid=(E, D // bd),
        in_specs=[
            pl.BlockSpec((1, SB, H2), lambda e, di, cnt_ref: (e, 0, 0)),
            pl.BlockSpec((1, bd, H), lambda e, di, cnt_ref: (e, di, 0)),
            pl.BlockSpec((1, SB, 128), lambda e, di, cnt_ref: (e, 0, 0)),
        ],
        out_specs=pl.BlockSpec((1, SB, bd), lambda e, di, cnt_ref: (e, 0, di)),
    )
    buf3 = buffer.reshape(E, SB, H2)
    sg3 = sg.reshape(E, SB, 128)
    return pl.pallas_call(
        _down_body,
        grid_spec=grid_spec,
        out_shape=jax.ShapeDtypeStruct((E, SB, D), jnp.float32),
    )(cnt, buf3, w_down, sg3)


def _combine_body(outbuf_hbm, idxc_hbm, y_hbm,
                  io_v, ra_v, rb_v, rc_v,
                  gsem0, gsem1, gsem2, wsem0, wsem1, wsem2):
    cid = lax.axis_index("c")
    sid = lax.axis_index("s")
    wid = sid * NC + cid
    pltpu.sync_copy(idxc_hbm.at[wid], io_v)
    rs = (ra_v, rb_v, rc_v)
    gsems = (gsem0, gsem1, gsem2)
    wsems = (wsem0, wsem1, wsem2)
    gets = [None, None, None]
    puts = [None, None, None]

    def start_gather(g):
        b = g % 3
        if puts[b] is not None:
            puts[b].wait()
            puts[b] = None
        gets[b] = pltpu.async_copy(
            outbuf_hbm.at[io_v.at[g]], rs[b], gsems[b])

    start_gather(0)
    start_gather(1)
    for g in range(NGD):
        b = g % 3
        if g + 2 < NGD:
            start_gather(g + 2)
        gets[b].wait()
        r_v = rs[b]

        def body(q, _):
            off = q * 16
            for t in range(16):
                r_v[t, pl.ds(off, 16)] = (
                    r_v[t, pl.ds(off, 16)] + r_v[16 + t, pl.ds(off, 16)])
            return 0

        lax.fori_loop(0, D // 16, body, 0)
        puts[b] = pltpu.async_copy(
            r_v.at[pl.ds(0, 16)],
            y_hbm.at[pl.ds(wid * TPW + g * 16, 16)], wsems[b])
    for p in puts:
        if p is not None:
            p.wait()


def _combine_call(out_flat, idxc):
    comb = functools.partial(
        pl.kernel,
        out_type=jax.ShapeDtypeStruct((N, D), jnp.float32),
        mesh=plsc.VectorSubcoreMesh(core_axis_name="c", subcore_axis_name="s"),
        compiler_params=pltpu.CompilerParams(needs_layout_passes=False),
        scratch_types=[
            pltpu.VMEM((NGD, 2 * 16), jnp.int32),
            pltpu.VMEM((2 * 16, D), jnp.float32),
            pltpu.VMEM((2 * 16, D), jnp.float32),
            pltpu.VMEM((2 * 16, D), jnp.float32),
            pltpu.SemaphoreType.DMA,
            pltpu.SemaphoreType.DMA,
            pltpu.SemaphoreType.DMA,
            pltpu.SemaphoreType.DMA,
            pltpu.SemaphoreType.DMA,
            pltpu.SemaphoreType.DMA,
        ],
    )(_combine_body)
    return comb(out_flat, idxc)


def kernel(x, W_router, W_gate, W_down):
    B, T, _ = x.shape
    x_flat = x.reshape(N, D)
    logitsT = _logits_call(x_flat, W_router)
    idxd, idxc, sg, cnt = _route_call(logitsT)
    hidden = _hidden_call(x_flat, W_gate)
    buffer = _dispatch_call(hidden, idxd)
    out_buf = _down_call(cnt, buffer, sg, W_down)
    y = _combine_call(out_buf.reshape(E * SB, D), idxc)
    return y.reshape(B, T, D)
